# baseline (speedup 1.0000x reference)
import functools

import jax
import jax.numpy as jnp
from jax import lax
from jax.experimental import pallas as pl
from jax.experimental.pallas import tpu as pltpu
from jax.experimental.pallas import tpu_sc as plsc

N_TOK = 2048
D_IN = 1024
P = 1024
H = 512
E = 64
K = 2
O = 512
CAP = 96
TB = 128
NB = N_TOK // TB
TRASH = E * CAP
BUF_ROWS = E * CAP + CAP
NSLOT = N_TOK * K

_NC = 2
_NS = 16
_NW = _NC * _NS
_SLOTS_PER_W = NSLOT // _NW
_CHUNK = 32


def _silu(v):
    return v / (1.0 + jnp.exp(-v))


def _stage_a_body(x_ref, win_ref, bin_ref, wg_ref,
                  h_ref, gf_ref, dst_ref, srcg_ref, carry_ref):
    @pl.when(pl.program_id(0) == 0)
    def _init():
        carry_ref[...] = jnp.zeros_like(carry_ref)

    xb = x_ref[...]
    hb = jnp.dot(xb, win_ref[...], preferred_element_type=jnp.float32)
    hb = hb + bin_ref[...]
    h_ref[...] = hb

    logits = jnp.dot(hb, wg_ref[...], preferred_element_type=jnp.float32)
    colf = lax.broadcasted_iota(jnp.int32, (TB, E), 1).astype(jnp.float32)
    m1 = jnp.max(logits, axis=1, keepdims=True)
    i1f = jnp.min(jnp.where(logits == m1, colf, 1e9), axis=1, keepdims=True)
    oh1 = colf == i1f
    lm = jnp.where(oh1, -1e30, logits)
    m2 = jnp.max(lm, axis=1, keepdims=True)
    i2f = jnp.min(jnp.where(lm == m2, colf, 1e9), axis=1, keepdims=True)
    oh2 = colf == i2f
    g1 = 1.0 / (1.0 + jnp.exp(m2 - m1))
    g2 = 1.0 / (1.0 + jnp.exp(m1 - m2))

    oh = oh1.astype(jnp.float32) + oh2.astype(jnp.float32)
    r = lax.broadcasted_iota(jnp.int32, (TB, TB), 0)
    c = lax.broadcasted_iota(jnp.int32, (TB, TB), 1)
    tril = (r > c).astype(jnp.bfloat16)
    cnt_prev = jnp.dot(tril, oh.astype(jnp.bfloat16),
                       preferred_element_type=jnp.float32)
    base = carry_ref[...] + cnt_prev
    pos1 = jnp.sum(jnp.where(oh1, base, 0.0), axis=1, keepdims=True)
    pos2 = jnp.sum(jnp.where(oh2, base, 0.0), axis=1, keepdims=True)
    carry_ref[...] = carry_ref[...] + jnp.sum(oh, axis=0, keepdims=True)

    k1 = pos1 < CAP
    k2 = pos2 < CAP
    i1 = i1f.astype(jnp.int32)
    i2 = i2f.astype(jnp.int32)
    p1 = jnp.clip(pos1.astype(jnp.int32), 0, CAP - 1)
    p2 = jnp.clip(pos2.astype(jnp.int32), 0, CAP - 1)
    row1 = i1 * CAP + p1
    row2 = i2 * CAP + p2
    d1 = jnp.where(k1, row1, TRASH + i1)
    d2 = jnp.where(k2, row2, TRASH + i2)
    s1 = jnp.where(k1, row1, 0)
    s2 = jnp.where(k2, row2, 0)
    gf_ref[...] = jnp.concatenate(
        [jnp.where(k1, g1, 0.0), jnp.where(k2, g2, 0.0)], axis=1)
    dst_ref[...] = jnp.concatenate([d1, d2], axis=1)
    srcg_ref[...] = jnp.concatenate([s1, s2], axis=1)


def _stage_a(x, w_in, b_in2, wg):
    return pl.pallas_call(
        _stage_a_body,
        grid=(NB,),
        in_specs=[
            pl.BlockSpec((TB, D_IN), lambda i: (i, 0)),
            pl.BlockSpec((D_IN, P), lambda i: (0, 0)),
            pl.BlockSpec((1, P), lambda i: (0, 0)),
            pl.BlockSpec((P, E), lambda i: (0, 0)),
        ],
        out_specs=[
            pl.BlockSpec((TB, P), lambda i: (i, 0)),
            pl.BlockSpec((TB, K), lambda i: (i, 0)),
            pl.BlockSpec((TB, K), lambda i: (i, 0)),
            pl.BlockSpec((TB, K), lambda i: (i, 0)),
        ],
        out_shape=[
            jax.ShapeDtypeStruct((N_TOK, P), jnp.float32),
            jax.ShapeDtypeStruct((N_TOK, K), jnp.float32),
            jax.ShapeDtypeStruct((N_TOK, K), jnp.int32),
            jax.ShapeDtypeStruct((N_TOK, K), jnp.int32),
        ],
        scratch_shapes=[pltpu.VMEM((1, E), jnp.float32)],
    )(x, w_in, b_in2, wg)


def _worker_id():
    return lax.axis_index("s") * _NC + lax.axis_index("c")


_NCH = _SLOTS_PER_W // _CHUNK


@functools.cache
def _build_sc_dispatch():
    mesh = plsc.VectorSubcoreMesh(core_axis_name="c", subcore_axis_name="s")

    @functools.partial(
        pl.kernel,
        out_type=jax.ShapeDtypeStruct((BUF_ROWS, P), jnp.float32),
        mesh=mesh,
        scratch_types=[
            pltpu.VMEM((_NCH, _CHUNK), jnp.int32),
            pltpu.VMEM((_NCH, _CHUNK), jnp.int32),
            pltpu.VMEM((_CHUNK, P), jnp.float32),
            pltpu.VMEM((_CHUNK, P), jnp.float32),
            pltpu.SemaphoreType.DMA,
            pltpu.SemaphoreType.DMA,
            pltpu.SemaphoreType.DMA,
            pltpu.SemaphoreType.DMA,
        ],
    )
    def dispatch(h_hbm, srctok_hbm, dst_hbm, buf_hbm,
                 st_v, dst_v, r0, r1, g0, g1, s0, s1):
        wid = _worker_id()
        rows = [r0, r1]
        gsem = [g0, g1]
        ssem = [s0, s1]
        pltpu.sync_copy(srctok_hbm.at[wid], st_v)
        pltpu.sync_copy(dst_hbm.at[wid], dst_v)
        gathers = [None, None]
        scatters = [None, None]
        gathers[0] = pltpu.async_copy(h_hbm.at[st_v.at[0]], rows[0], gsem[0])
        for c in range(_NCH):
            b = c % 2
            nb = (c + 1) % 2
            if c + 1 < _NCH:
                if scatters[nb] is not None:
                    scatters[nb].wait()
                gathers[nb] = pltpu.async_copy(
                    h_hbm.at[st_v.at[c + 1]], rows[nb], gsem[nb])
            gathers[b].wait()
            scatters[b] = pltpu.async_copy(
                rows[b], buf_hbm.at[dst_v.at[c]], ssem[b])
        scatters[(_NCH - 1) % 2].wait()
        scatters[_NCH % 2].wait()

    return dispatch


@functools.cache
def _build_sc_combine():
    mesh = plsc.VectorSubcoreMesh(core_axis_name="c", subcore_axis_name="s")

    @functools.partial(
        pl.kernel,
        out_type=jax.ShapeDtypeStruct((NSLOT, P), jnp.float32),
        mesh=mesh,
        scratch_types=[
            pltpu.VMEM((_NCH, _CHUNK), jnp.int32),
            pltpu.VMEM((_CHUNK, P), jnp.float32),
            pltpu.VMEM((_CHUNK, P), jnp.float32),
            pltpu.SemaphoreType.DMA,
            pltpu.SemaphoreType.DMA,
            pltpu.SemaphoreType.DMA,
            pltpu.SemaphoreType.DMA,
        ],
    )
    def combine(eout_hbm, srcg_hbm, yrep_hbm, idx_v, r0, r1, g0, g1, s0, s1):
        wid = _worker_id()
        rows = [r0, r1]
        gsem = [g0, g1]
        ssem = [s0, s1]
        pltpu.sync_copy(srcg_hbm.at[wid], idx_v)
        gathers = [None, None]
        stores = [None, None]
        gathers[0] = pltpu.async_copy(eout_hbm.at[idx_v.at[0]], rows[0], gsem[0])
        for c in range(_NCH):
            b = c % 2
            nb = (c + 1) % 2
            if c + 1 < _NCH:
                if stores[nb] is not None:
                    stores[nb].wait()
                gathers[nb] = pltpu.async_copy(
                    eout_hbm.at[idx_v.at[c + 1]], rows[nb], gsem[nb])
            gathers[b].wait()
            base = wid * _SLOTS_PER_W + c * _CHUNK
            stores[b] = pltpu.async_copy(
                rows[b], yrep_hbm.at[pl.ds(base, _CHUNK)], ssem[b])
        stores[(_NCH - 1) % 2].wait()
        stores[_NCH % 2].wait()

    return combine


def _sc_dispatch(h, src_tok, dst_f):
    st3 = src_tok.reshape(_NW, _NCH, _CHUNK)
    dst3 = dst_f.reshape(_NW, _NCH, _CHUNK)
    return _build_sc_dispatch()(h, st3, dst3)


def _sc_combine(eout, srcg_f):
    sg3 = srcg_f.reshape(_NW, _NCH, _CHUNK)
    return _build_sc_combine()(eout, sg3)


_EB = 2


def _stage_d_body(buf_ref, w1_ref, b1_ref, w2_ref, b2_ref, eout_ref):
    for j in range(_EB):
        rows = buf_ref[pl.ds(j * CAP, CAP), :]
        a = jnp.dot(rows, w1_ref[j], preferred_element_type=jnp.float32)
        a = _silu(a + b1_ref[j])
        o = jnp.dot(a, w2_ref[j], preferred_element_type=jnp.float32)
        eout_ref[pl.ds(j * CAP, CAP), :] = o + b2_ref[j]


def _stage_d(buf, w1, b1, w2, b2):
    return pl.pallas_call(
        _stage_d_body,
        grid=(E // _EB,),
        in_specs=[
            pl.BlockSpec((_EB * CAP, P), lambda e: (e, 0)),
            pl.BlockSpec((_EB, P, H), lambda e: (e, 0, 0)),
            pl.BlockSpec((_EB, 1, H), lambda e: (e, 0, 0)),
            pl.BlockSpec((_EB, H, P), lambda e: (e, 0, 0)),
            pl.BlockSpec((_EB, 1, P), lambda e: (e, 0, 0)),
        ],
        out_specs=pl.BlockSpec((_EB * CAP, P), lambda e: (e, 0)),
        out_shape=jax.ShapeDtypeStruct((E * CAP, P), jnp.float32),
    )(buf, w1, b1.reshape(E, 1, H), w2, b2.reshape(E, 1, P))


def _shared_body(h_ref, wsg_ref, bsg_ref, wsu_ref, bsu_ref, wsd_ref, bsd_ref,
                 sh_ref):
    hb = h_ref[...]
    sg = jnp.dot(hb, wsg_ref[...], preferred_element_type=jnp.float32)
    su = jnp.dot(hb, wsu_ref[...], preferred_element_type=jnp.float32)
    mid = _silu(sg + bsg_ref[...]) * (su + bsu_ref[...])
    sh = jnp.dot(mid, wsd_ref[...], preferred_element_type=jnp.float32)
    sh_ref[...] = sh + bsd_ref[...]


def _stage_shared(h, wsg, bsg2, wsu, bsu2, wsd, bsd2):
    full = lambda shape: pl.BlockSpec(shape, lambda i: tuple(0 for _ in shape))
    return pl.pallas_call(
        _shared_body,
        grid=(NB,),
        in_specs=[
            pl.BlockSpec((TB, P), lambda i: (i, 0)),
            full((P, H)), full((1, H)),
            full((P, H)), full((1, H)),
            full((H, P)), full((1, P)),
        ],
        out_specs=pl.BlockSpec((TB, P), lambda i: (i, 0)),
        out_shape=jax.ShapeDtypeStruct((N_TOK, P), jnp.float32),
    )(h, wsg, bsg2, wsu, bsu2, wsd, bsd2)


def _stage_f_body(y0_ref, y1_ref, g0_ref, g1_ref, sh_ref,
                  wm1_ref, bm1_ref, wm2_ref, bm2_ref, wo_ref, bo_ref,
                  out_ref):
    g0 = g0_ref[...]
    g1 = g1_ref[...]
    moe = (jnp.where(g0 > 0, y0_ref[...] * g0, 0.0)
           + jnp.where(g1 > 0, y1_ref[...] * g1, 0.0))
    y = moe + sh_ref[...]
    t = _silu(jnp.dot(y, wm1_ref[...], preferred_element_type=jnp.float32)
              + bm1_ref[...])
    t = jnp.dot(t, wm2_ref[...], preferred_element_type=jnp.float32)
    t = t + bm2_ref[...]
    out_ref[...] = (jnp.dot(t, wo_ref[...], preferred_element_type=jnp.float32)
                    + bo_ref[...])


def _stage_f(yrep, gf_col, shared, wm1, bm12, wm2, bm22, wo, bo2):
    full = lambda shape: pl.BlockSpec(shape, lambda i: tuple(0 for _ in shape))
    return pl.pallas_call(
        _stage_f_body,
        grid=(NB,),
        in_specs=[
            pl.BlockSpec((TB, P), lambda i: (i, 0)),
            pl.BlockSpec((TB, P), lambda i: (NB + i, 0)),
            pl.BlockSpec((TB, 1), lambda i: (i, 0)),
            pl.BlockSpec((TB, 1), lambda i: (NB + i, 0)),
            pl.BlockSpec((TB, P), lambda i: (i, 0)),
            full((P, H)), full((1, H)),
            full((H, H)), full((1, H)),
            full((H, O)), full((1, O)),
        ],
        out_specs=pl.BlockSpec((TB, O), lambda i: (i, 0)),
        out_shape=jax.ShapeDtypeStruct((N_TOK, O), jnp.float32),
    )(yrep, yrep, gf_col, gf_col, shared, wm1, bm12, wm2, bm22, wo, bo2)


def kernel(x, W_in, b_in, Wg, W1, b1, W2, b2, Wsg, bsg, Wsu, bsu,
           Wsd, bsd, Wm1, bm1, Wm2, bm2, Wo, bo):
    h, gf, dst, srcg = _stage_a(x, W_in, b_in.reshape(1, P), Wg)

    dst_f = dst.T.reshape(-1)
    srcg_f = srcg.T.reshape(-1)
    gf_col = gf.T.reshape(-1, 1)
    src_tok = jnp.tile(jnp.arange(N_TOK, dtype=jnp.int32), K)

    buf = _sc_dispatch(h, src_tok, dst_f)
    shared = _stage_shared(h, Wsg, bsg.reshape(1, H), Wsu, bsu.reshape(1, H),
                           Wsd, bsd.reshape(1, P))
    eout = _stage_d(buf, W1, b1, W2, b2)
    yrep = _sc_combine(eout, srcg_f)

    return _stage_f(
        yrep, gf_col, shared,
        Wm1, bm1.reshape(1, H), Wm2, bm2.reshape(1, H), Wo, bo.reshape(1, O))


# speedup vs baseline: 1.0432x; 1.0432x over previous
import functools

import jax
import jax.numpy as jnp
from jax import lax
from jax.experimental import pallas as pl
from jax.experimental.pallas import tpu as pltpu
from jax.experimental.pallas import tpu_sc as plsc

N_TOK = 2048
D_IN = 1024
P = 1024
H = 512
E = 64
K = 2
O = 512
CAP = 96
TB = 128
NB = N_TOK // TB
TRASH = E * CAP
BUF_ROWS = E * CAP + CAP
NSLOT = N_TOK * K

_NC = 2
_NS = 16
_NW = _NC * _NS
_SLOTS_PER_W = NSLOT // _NW
_CHUNK = 32


def _silu(v):
    return v / (1.0 + jnp.exp(-v))


def _stage_a_body(x_ref, win_ref, bin_ref, wg_ref,
                  h_ref, gf_ref, dst_ref, srcg_ref, carry_ref):
    @pl.when(pl.program_id(0) == 0)
    def _init():
        carry_ref[...] = jnp.zeros_like(carry_ref)

    xb = x_ref[...]
    hb = jnp.dot(xb, win_ref[...], preferred_element_type=jnp.float32)
    hb = hb + bin_ref[...]
    h_ref[...] = hb

    logits = jnp.dot(hb, wg_ref[...], preferred_element_type=jnp.float32)
    colf = lax.broadcasted_iota(jnp.int32, (TB, E), 1).astype(jnp.float32)
    m1 = jnp.max(logits, axis=1, keepdims=True)
    i1f = jnp.min(jnp.where(logits == m1, colf, 1e9), axis=1, keepdims=True)
    oh1 = colf == i1f
    lm = jnp.where(oh1, -1e30, logits)
    m2 = jnp.max(lm, axis=1, keepdims=True)
    i2f = jnp.min(jnp.where(lm == m2, colf, 1e9), axis=1, keepdims=True)
    oh2 = colf == i2f
    g1 = 1.0 / (1.0 + jnp.exp(m2 - m1))
    g2 = 1.0 / (1.0 + jnp.exp(m1 - m2))

    oh = oh1.astype(jnp.float32) + oh2.astype(jnp.float32)
    r = lax.broadcasted_iota(jnp.int32, (TB, TB), 0)
    c = lax.broadcasted_iota(jnp.int32, (TB, TB), 1)
    tril = (r > c).astype(jnp.bfloat16)
    cnt_prev = jnp.dot(tril, oh.astype(jnp.bfloat16),
                       preferred_element_type=jnp.float32)
    base = carry_ref[...] + cnt_prev
    pos1 = jnp.sum(jnp.where(oh1, base, 0.0), axis=1, keepdims=True)
    pos2 = jnp.sum(jnp.where(oh2, base, 0.0), axis=1, keepdims=True)
    carry_ref[...] = carry_ref[...] + jnp.sum(oh, axis=0, keepdims=True)

    k1 = pos1 < CAP
    k2 = pos2 < CAP
    i1 = i1f.astype(jnp.int32)
    i2 = i2f.astype(jnp.int32)
    p1 = jnp.clip(pos1.astype(jnp.int32), 0, CAP - 1)
    p2 = jnp.clip(pos2.astype(jnp.int32), 0, CAP - 1)
    row1 = i1 * CAP + p1
    row2 = i2 * CAP + p2
    d1 = jnp.where(k1, row1, TRASH + i1)
    d2 = jnp.where(k2, row2, TRASH + i2)
    s1 = jnp.where(k1, row1, 0)
    s2 = jnp.where(k2, row2, 0)
    gf_ref[...] = jnp.concatenate(
        [jnp.where(k1, g1, 0.0), jnp.where(k2, g2, 0.0)], axis=1)
    dst_ref[...] = jnp.concatenate([d1, d2], axis=1)
    srcg_ref[...] = jnp.concatenate([s1, s2], axis=1)


def _stage_a(x, w_in, b_in2, wg):
    return pl.pallas_call(
        _stage_a_body,
        grid=(NB,),
        in_specs=[
            pl.BlockSpec((TB, D_IN), lambda i: (i, 0)),
            pl.BlockSpec((D_IN, P), lambda i: (0, 0)),
            pl.BlockSpec((1, P), lambda i: (0, 0)),
            pl.BlockSpec((P, E), lambda i: (0, 0)),
        ],
        out_specs=[
            pl.BlockSpec((TB, P), lambda i: (i, 0)),
            pl.BlockSpec((TB, K), lambda i: (i, 0)),
            pl.BlockSpec((TB, K), lambda i: (i, 0)),
            pl.BlockSpec((TB, K), lambda i: (i, 0)),
        ],
        out_shape=[
            jax.ShapeDtypeStruct((N_TOK, P), jnp.float32),
            jax.ShapeDtypeStruct((N_TOK, K), jnp.float32),
            jax.ShapeDtypeStruct((N_TOK, K), jnp.int32),
            jax.ShapeDtypeStruct((N_TOK, K), jnp.int32),
        ],
        scratch_shapes=[pltpu.VMEM((1, E), jnp.float32)],
    )(x, w_in, b_in2, wg)


def _worker_id():
    return lax.axis_index("s") * _NC + lax.axis_index("c")


_NCH = _SLOTS_PER_W // _CHUNK


@functools.cache
def _build_sc_dispatch():
    mesh = plsc.VectorSubcoreMesh(core_axis_name="c", subcore_axis_name="s")

    @functools.partial(
        pl.kernel,
        out_type=jax.ShapeDtypeStruct((BUF_ROWS, P), jnp.float32),
        mesh=mesh,
        scratch_types=[
            pltpu.VMEM((_NCH, _CHUNK), jnp.int32),
            pltpu.VMEM((_CHUNK, P), jnp.float32),
            pltpu.VMEM((_CHUNK, P), jnp.float32),
            pltpu.SemaphoreType.DMA,
            pltpu.SemaphoreType.DMA,
            pltpu.SemaphoreType.DMA,
            pltpu.SemaphoreType.DMA,
        ],
    )
    def dispatch(h_hbm, dst_hbm, buf_hbm, dst_v, r0, r1, g0, g1, s0, s1):
        wid = _worker_id()
        rows = [r0, r1]
        gsem = [g0, g1]
        ssem = [s0, s1]
        pltpu.sync_copy(dst_hbm.at[wid], dst_v)
        tok0 = (wid * _SLOTS_PER_W) % N_TOK
        gathers = [None, None]
        scatters = [None, None]
        gathers[0] = pltpu.async_copy(
            h_hbm.at[pl.ds(tok0, _CHUNK)], rows[0], gsem[0])
        for c in range(_NCH):
            b = c % 2
            nb = (c + 1) % 2
            if c + 1 < _NCH:
                if scatters[nb] is not None:
                    scatters[nb].wait()
                gathers[nb] = pltpu.async_copy(
                    h_hbm.at[pl.ds(tok0 + (c + 1) * _CHUNK, _CHUNK)],
                    rows[nb], gsem[nb])
            gathers[b].wait()
            scatters[b] = pltpu.async_copy(
                rows[b], buf_hbm.at[dst_v.at[c]], ssem[b])
        scatters[(_NCH - 1) % 2].wait()
        scatters[_NCH % 2].wait()

    return dispatch


@functools.cache
def _build_sc_combine():
    mesh = plsc.VectorSubcoreMesh(core_axis_name="c", subcore_axis_name="s")

    @functools.partial(
        pl.kernel,
        out_type=jax.ShapeDtypeStruct((NSLOT, P), jnp.float32),
        mesh=mesh,
        scratch_types=[
            pltpu.VMEM((_NCH, _CHUNK), jnp.int32),
            pltpu.VMEM((_CHUNK, P), jnp.float32),
            pltpu.VMEM((_CHUNK, P), jnp.float32),
            pltpu.SemaphoreType.DMA,
            pltpu.SemaphoreType.DMA,
            pltpu.SemaphoreType.DMA,
            pltpu.SemaphoreType.DMA,
        ],
    )
    def combine(eout_hbm, srcg_hbm, yrep_hbm, idx_v, r0, r1, g0, g1, s0, s1):
        wid = _worker_id()
        rows = [r0, r1]
        gsem = [g0, g1]
        ssem = [s0, s1]
        pltpu.sync_copy(srcg_hbm.at[wid], idx_v)
        gathers = [None, None]
        stores = [None, None]
        gathers[0] = pltpu.async_copy(eout_hbm.at[idx_v.at[0]], rows[0], gsem[0])
        for c in range(_NCH):
            b = c % 2
            nb = (c + 1) % 2
            if c + 1 < _NCH:
                if stores[nb] is not None:
                    stores[nb].wait()
                gathers[nb] = pltpu.async_copy(
                    eout_hbm.at[idx_v.at[c + 1]], rows[nb], gsem[nb])
            gathers[b].wait()
            base = wid * _SLOTS_PER_W + c * _CHUNK
            stores[b] = pltpu.async_copy(
                rows[b], yrep_hbm.at[pl.ds(base, _CHUNK)], ssem[b])
        stores[(_NCH - 1) % 2].wait()
        stores[_NCH % 2].wait()

    return combine


def _sc_dispatch(h, dst_f):
    dst3 = dst_f.reshape(_NW, _NCH, _CHUNK)
    return _build_sc_dispatch()(h, dst3)


def _sc_combine(eout, srcg_f):
    sg3 = srcg_f.reshape(_NW, _NCH, _CHUNK)
    return _build_sc_combine()(eout, sg3)


_EB = 2


def _stage_d_body(buf_ref, w1_ref, b1_ref, w2_ref, b2_ref, eout_ref):
    for j in range(_EB):
        rows = buf_ref[pl.ds(j * CAP, CAP), :]
        a = jnp.dot(rows, w1_ref[j], preferred_element_type=jnp.float32)
        a = _silu(a + b1_ref[j])
        o = jnp.dot(a, w2_ref[j], preferred_element_type=jnp.float32)
        eout_ref[pl.ds(j * CAP, CAP), :] = o + b2_ref[j]


def _stage_d(buf, w1, b1, w2, b2):
    return pl.pallas_call(
        _stage_d_body,
        grid=(E // _EB,),
        in_specs=[
            pl.BlockSpec((_EB * CAP, P), lambda e: (e, 0)),
            pl.BlockSpec((_EB, P, H), lambda e: (e, 0, 0)),
            pl.BlockSpec((_EB, 1, H), lambda e: (e, 0, 0)),
            pl.BlockSpec((_EB, H, P), lambda e: (e, 0, 0)),
            pl.BlockSpec((_EB, 1, P), lambda e: (e, 0, 0)),
        ],
        out_specs=pl.BlockSpec((_EB * CAP, P), lambda e: (e, 0)),
        out_shape=jax.ShapeDtypeStruct((E * CAP, P), jnp.float32),
    )(buf, w1, b1.reshape(E, 1, H), w2, b2.reshape(E, 1, P))


def _stage_f_body(y0_ref, y1_ref, g0_ref, g1_ref, h_ref,
                  wsg_ref, bsg_ref, wsu_ref, bsu_ref, wsd_ref, bsd_ref,
                  wm1_ref, bm1_ref, wm2_ref, bm2_ref, wo_ref, bo_ref,
                  out_ref):
    g0 = g0_ref[...]
    g1 = g1_ref[...]
    moe = (jnp.where(g0 > 0, y0_ref[...] * g0, 0.0)
           + jnp.where(g1 > 0, y1_ref[...] * g1, 0.0))
    hb = h_ref[...]
    sg = jnp.dot(hb, wsg_ref[...], preferred_element_type=jnp.float32)
    su = jnp.dot(hb, wsu_ref[...], preferred_element_type=jnp.float32)
    mid = _silu(sg + bsg_ref[...]) * (su + bsu_ref[...])
    shared = jnp.dot(mid, wsd_ref[...], preferred_element_type=jnp.float32)
    y = moe + shared + bsd_ref[...]
    t = _silu(jnp.dot(y, wm1_ref[...], preferred_element_type=jnp.float32)
              + bm1_ref[...])
    t = jnp.dot(t, wm2_ref[...], preferred_element_type=jnp.float32)
    t = t + bm2_ref[...]
    out_ref[...] = (jnp.dot(t, wo_ref[...], preferred_element_type=jnp.float32)
                    + bo_ref[...])


def _stage_f(yrep, gf_col, h, wsg, bsg2, wsu, bsu2, wsd, bsd2,
             wm1, bm12, wm2, bm22, wo, bo2):
    full = lambda shape: pl.BlockSpec(shape, lambda i: tuple(0 for _ in shape))
    return pl.pallas_call(
        _stage_f_body,
        grid=(NB,),
        in_specs=[
            pl.BlockSpec((TB, P), lambda i: (i, 0)),
            pl.BlockSpec((TB, P), lambda i: (NB + i, 0)),
            pl.BlockSpec((TB, 1), lambda i: (i, 0)),
            pl.BlockSpec((TB, 1), lambda i: (NB + i, 0)),
            pl.BlockSpec((TB, P), lambda i: (i, 0)),
            full((P, H)), full((1, H)),
            full((P, H)), full((1, H)),
            full((H, P)), full((1, P)),
            full((P, H)), full((1, H)),
            full((H, H)), full((1, H)),
            full((H, O)), full((1, O)),
        ],
        out_specs=pl.BlockSpec((TB, O), lambda i: (i, 0)),
        out_shape=jax.ShapeDtypeStruct((N_TOK, O), jnp.float32),
    )(yrep, yrep, gf_col, gf_col, h, wsg, bsg2, wsu, bsu2, wsd, bsd2,
      wm1, bm12, wm2, bm22, wo, bo2)


def kernel(x, W_in, b_in, Wg, W1, b1, W2, b2, Wsg, bsg, Wsu, bsu,
           Wsd, bsd, Wm1, bm1, Wm2, bm2, Wo, bo):
    h, gf, dst, srcg = _stage_a(x, W_in, b_in.reshape(1, P), Wg)

    dst_f = dst.T.reshape(-1)
    srcg_f = srcg.T.reshape(-1)
    gf_col = gf.T.reshape(-1, 1)

    buf = _sc_dispatch(h, dst_f)
    eout = _stage_d(buf, W1, b1, W2, b2)
    yrep = _sc_combine(eout, srcg_f)

    return _stage_f(
        yrep, gf_col, h,
        Wsg, bsg.reshape(1, H), Wsu, bsu.reshape(1, H), Wsd, bsd.reshape(1, P),
        Wm1, bm1.reshape(1, H), Wm2, bm2.reshape(1, H), Wo, bo.reshape(1, O))
